# Initial kernel scaffold; baseline (speedup 1.0000x reference)
import jax
import jax.numpy as jnp
from jax.experimental import pallas as pl


def kernel(x, A, mask, weight, bias):
    raise NotImplementedError("write your pallas kernel here")


# speedup vs baseline: 3.1247x; 3.1247x over previous
import functools

import jax
import jax.numpy as jnp
from jax.experimental import pallas as pl
from jax.experimental.pallas import tpu as pltpu

INC = 128
INN = 512
OUTC = 64
OUTN = 64
MAXD = 8
N = 512

BN = 8


def _body(a_ref, m_ref, w_ref, b_ref, x_ref, o_ref, p_scr):
    @pl.when(pl.program_id(0) == 0)
    def _():
        rows = jax.lax.broadcasted_iota(jnp.int32, (INN, OUTN), 0)
        acc = jnp.zeros((INN, OUTN), jnp.float32)
        for d in range(MAXD):
            acc = acc + jnp.where(rows == a_ref[d : d + 1, :],
                                  m_ref[d : d + 1, :], 0.0)
        p_scr[...] = acc

    xb = x_ref[...].reshape(BN * INC, INN)
    xp = jnp.dot(xb, p_scr[...], preferred_element_type=jnp.float32)
    xp = xp.reshape(BN, INC, OUTN)
    yb = jax.lax.dot_general(xp, w_ref[...], (((1,), (0,)), ((), ())),
                             preferred_element_type=jnp.float32)
    o_ref[...] = jnp.transpose(yb, (0, 2, 1)) + b_ref[...][None, :, :]


@jax.jit
def kernel(x, A, mask, weight, bias):
    at = A.T.astype(jnp.int32)
    mt = mask[:, :, 0].T
    grid = (N // BN,)
    out = pl.pallas_call(
        _body,
        grid=grid,
        in_specs=[
            pl.BlockSpec((MAXD, OUTN), lambda i: (0, 0)),
            pl.BlockSpec((MAXD, OUTN), lambda i: (0, 0)),
            pl.BlockSpec((INC, OUTC), lambda i: (0, 0)),
            pl.BlockSpec((OUTC, 1), lambda i: (0, 0)),
            pl.BlockSpec((BN, INC, INN), lambda i: (i, 0, 0)),
        ],
        out_specs=pl.BlockSpec((BN, OUTC, OUTN), lambda i: (i, 0, 0)),
        out_shape=jax.ShapeDtypeStruct((N, OUTC, OUTN), jnp.float32),
        scratch_shapes=[pltpu.VMEM((INN, OUTN), jnp.float32)],
    )(at, mt, weight, bias, x)
    return out


# speedup vs baseline: 4.1185x; 1.3180x over previous
import functools

import jax
import jax.numpy as jnp
from jax.experimental import pallas as pl
from jax.experimental.pallas import tpu as pltpu

INC = 128
INN = 512
OUTC = 64
OUTN = 64
MAXD = 8
N = 512

BN = 16


def _body(a_ref, m_ref, w_ref, b_ref, x_ref, o_ref, p_scr):
    @pl.when(pl.program_id(0) == 0)
    def _():
        rows = jax.lax.broadcasted_iota(jnp.int32, (INN, OUTN), 0)
        acc = jnp.zeros((INN, OUTN), jnp.float32)
        for d in range(MAXD):
            acc = acc + jnp.where(rows == a_ref[d : d + 1, :],
                                  m_ref[d : d + 1, :], 0.0)
        p_scr[...] = acc

    xb = x_ref[...].reshape(BN * INC, INN)
    xp = jnp.dot(xb, p_scr[...], preferred_element_type=jnp.float32)
    xp = xp.reshape(BN, INC, OUTN)
    yb = jax.lax.dot_general(xp, w_ref[...], (((1,), (0,)), ((), ())),
                             preferred_element_type=jnp.float32)
    o_ref[...] = jnp.transpose(yb, (0, 2, 1)) + b_ref[...][None, :, :]


@jax.jit
def kernel(x, A, mask, weight, bias):
    at = A.T.astype(jnp.int32)
    mt = mask[:, :, 0].T
    grid = (N // BN,)
    out = pl.pallas_call(
        _body,
        grid=grid,
        in_specs=[
            pl.BlockSpec((MAXD, OUTN), lambda i: (0, 0)),
            pl.BlockSpec((MAXD, OUTN), lambda i: (0, 0)),
            pl.BlockSpec((INC, OUTC), lambda i: (0, 0)),
            pl.BlockSpec((OUTC, 1), lambda i: (0, 0)),
            pl.BlockSpec((BN, INC, INN), lambda i: (i, 0, 0)),
        ],
        out_specs=pl.BlockSpec((BN, OUTC, OUTN), lambda i: (i, 0, 0)),
        out_shape=jax.ShapeDtypeStruct((N, OUTC, OUTN), jnp.float32),
        scratch_shapes=[pltpu.VMEM((INN, OUTN), jnp.float32)],
    )(at, mt, weight, bias, x)
    return out


# speedup vs baseline: 4.7105x; 1.1437x over previous
import functools

import jax
import jax.numpy as jnp
from jax.experimental import pallas as pl
from jax.experimental.pallas import tpu as pltpu

INC = 128
INN = 512
OUTC = 64
OUTN = 64
MAXD = 8
N = 512

BN = 32


def _body(a_ref, m_ref, w_ref, b_ref, x_ref, o_ref, p_scr):
    @pl.when(pl.program_id(0) == 0)
    def _():
        rows = jax.lax.broadcasted_iota(jnp.int32, (INN, OUTN), 0)
        acc = jnp.zeros((INN, OUTN), jnp.float32)
        for d in range(MAXD):
            acc = acc + jnp.where(rows == a_ref[d : d + 1, :],
                                  m_ref[d : d + 1, :], 0.0)
        p_scr[...] = acc

    xb = x_ref[...].reshape(BN * INC, INN)
    xp = jnp.dot(xb, p_scr[...], preferred_element_type=jnp.float32)
    xp = xp.reshape(BN, INC, OUTN)
    yb = jax.lax.dot_general(xp, w_ref[...], (((1,), (0,)), ((), ())),
                             preferred_element_type=jnp.float32)
    o_ref[...] = jnp.transpose(yb, (0, 2, 1)) + b_ref[...][None, :, :]


@jax.jit
def kernel(x, A, mask, weight, bias):
    at = A.T.astype(jnp.int32)
    mt = mask[:, :, 0].T
    grid = (N // BN,)
    out = pl.pallas_call(
        _body,
        grid=grid,
        in_specs=[
            pl.BlockSpec((MAXD, OUTN), lambda i: (0, 0)),
            pl.BlockSpec((MAXD, OUTN), lambda i: (0, 0)),
            pl.BlockSpec((INC, OUTC), lambda i: (0, 0)),
            pl.BlockSpec((OUTC, 1), lambda i: (0, 0)),
            pl.BlockSpec((BN, INC, INN), lambda i: (i, 0, 0)),
        ],
        out_specs=pl.BlockSpec((BN, OUTC, OUTN), lambda i: (i, 0, 0)),
        out_shape=jax.ShapeDtypeStruct((N, OUTC, OUTN), jnp.float32),
        scratch_shapes=[pltpu.VMEM((INN, OUTN), jnp.float32)],
    )(at, mt, weight, bias, x)
    return out


# speedup vs baseline: 4.8405x; 1.0276x over previous
import functools

import jax
import jax.numpy as jnp
from jax.experimental import pallas as pl
from jax.experimental.pallas import tpu as pltpu

INC = 128
INN = 512
OUTC = 64
OUTN = 64
MAXD = 8
N = 512

BN = 64


def _body(a_ref, m_ref, w_ref, b_ref, x_ref, o_ref, p_scr):
    @pl.when(pl.program_id(0) == 0)
    def _():
        rows = jax.lax.broadcasted_iota(jnp.int32, (INN, OUTN), 0)
        acc = jnp.zeros((INN, OUTN), jnp.float32)
        for d in range(MAXD):
            acc = acc + jnp.where(rows == a_ref[d : d + 1, :],
                                  m_ref[d : d + 1, :], 0.0)
        p_scr[...] = acc

    xb = x_ref[...].reshape(BN * INC, INN)
    xp = jnp.dot(xb, p_scr[...], preferred_element_type=jnp.float32)
    xp = xp.reshape(BN, INC, OUTN)
    yb = jax.lax.dot_general(xp, w_ref[...], (((1,), (0,)), ((), ())),
                             preferred_element_type=jnp.float32)
    o_ref[...] = jnp.transpose(yb, (0, 2, 1)) + b_ref[...][None, :, :]


@jax.jit
def kernel(x, A, mask, weight, bias):
    at = A.T.astype(jnp.int32)
    mt = mask[:, :, 0].T
    grid = (N // BN,)
    out = pl.pallas_call(
        _body,
        grid=grid,
        in_specs=[
            pl.BlockSpec((MAXD, OUTN), lambda i: (0, 0)),
            pl.BlockSpec((MAXD, OUTN), lambda i: (0, 0)),
            pl.BlockSpec((INC, OUTC), lambda i: (0, 0)),
            pl.BlockSpec((OUTC, 1), lambda i: (0, 0)),
            pl.BlockSpec((BN, INC, INN), lambda i: (i, 0, 0)),
        ],
        out_specs=pl.BlockSpec((BN, OUTC, OUTN), lambda i: (i, 0, 0)),
        out_shape=jax.ShapeDtypeStruct((N, OUTC, OUTN), jnp.float32),
        scratch_shapes=[pltpu.VMEM((INN, OUTN), jnp.float32)],
    )(at, mt, weight, bias, x)
    return out


# speedup vs baseline: 4.8443x; 1.0008x over previous
import functools

import jax
import jax.numpy as jnp
from jax.experimental import pallas as pl
from jax.experimental.pallas import tpu as pltpu

INC = 128
INN = 512
OUTC = 64
OUTN = 64
MAXD = 8
N = 512

BN = 64


def _body(a_ref, m_ref, w_ref, b_ref, x_ref, o_ref, p_scr):
    @pl.when(pl.program_id(0) == 0)
    def _():
        rows = jax.lax.broadcasted_iota(jnp.int32, (INN, OUTN), 0)
        acc = jnp.zeros((INN, OUTN), jnp.float32)
        for d in range(MAXD):
            acc = acc + jnp.where(rows == a_ref[d : d + 1, :],
                                  m_ref[d : d + 1, :], 0.0)
        p_scr[...] = acc.astype(jnp.bfloat16)

    xb = x_ref[...].reshape(BN * INC, INN).astype(jnp.bfloat16)
    xp = jnp.dot(xb, p_scr[...], preferred_element_type=jnp.float32)
    xp = xp.reshape(BN, INC, OUTN)
    yb = jax.lax.dot_general(xp, w_ref[...], (((1,), (0,)), ((), ())),
                             preferred_element_type=jnp.float32)
    o_ref[...] = jnp.transpose(yb, (0, 2, 1)) + b_ref[...][None, :, :]


@jax.jit
def kernel(x, A, mask, weight, bias):
    at = A.T.astype(jnp.int32)
    mt = mask[:, :, 0].T
    grid = (N // BN,)
    out = pl.pallas_call(
        _body,
        grid=grid,
        in_specs=[
            pl.BlockSpec((MAXD, OUTN), lambda i: (0, 0)),
            pl.BlockSpec((MAXD, OUTN), lambda i: (0, 0)),
            pl.BlockSpec((INC, OUTC), lambda i: (0, 0)),
            pl.BlockSpec((OUTC, 1), lambda i: (0, 0)),
            pl.BlockSpec((BN, INC, INN), lambda i: (i, 0, 0)),
        ],
        out_specs=pl.BlockSpec((BN, OUTC, OUTN), lambda i: (i, 0, 0)),
        out_shape=jax.ShapeDtypeStruct((N, OUTC, OUTN), jnp.float32),
        scratch_shapes=[pltpu.VMEM((INN, OUTN), jnp.bfloat16)],
    )(at, mt, weight, bias, x)
    return out
